# baseline (speedup 1.0000x reference)
import functools

import jax
import jax.numpy as jnp
from jax import lax
from jax.experimental import pallas as pl
from jax.experimental.pallas import tpu as pltpu
from jax.experimental.pallas import tpu_sc as plsc

N = 10000
E = 160000
D_IN = 256
D_H = 512
NG = 16

NT = 16
NPAD = 10240
RPT = NPAD // NT
K = 128
STEPS = 80
EPT = K * STEPS
EPAD = NT * EPT
RB = 256
GRID = NPAD // RB


def _make_sc_agg(n_chunks):
    cps = n_chunks // 2
    mesh = plsc.VectorSubcoreMesh(core_axis_name="c", subcore_axis_name="s",
                                  num_cores=2, num_subcores=NT)
    out_type = [jax.ShapeDtypeStruct((n_chunks, NPAD, 128), jnp.float32)]
    scratch = [
        pltpu.VMEM((STEPS, K), jnp.int32),
        pltpu.VMEM((K,), jnp.int32),
        pltpu.VMEM((K,), jnp.int32),
        pltpu.VMEM((K, 128), jnp.float32),
        pltpu.VMEM((K, 128), jnp.float32),
        pltpu.VMEM_SHARED((NPAD, 128), jnp.float32),
        pltpu.SemaphoreType.DMA,
        pltpu.SemaphoreType.DMA,
        pltpu.SemaphoreType.DMA,
        pltpu.SemaphoreType.DMA,
        pltpu.SemaphoreType.DMA,
        pltpu.SemaphoreType.DMA,
    ]

    def body(y, srcr, dstr, z128, out,
             idx_d, isb0, isb1, rows0, rows1, aggsh,
             sg0, sg1, ss0, ss1, si0, si1):
        c = lax.axis_index("c")
        s = lax.axis_index("s")
        r0 = s * RPT
        T = STEPS // 2
        pltpu.sync_copy(dstr.at[s], idx_d)
        for qi in range(cps):
            q = c * cps + qi
            pltpu.sync_copy(z128.at[pl.ds(r0, RPT)], aggsh.at[pl.ds(r0, RPT)])
            pltpu.async_copy(srcr.at[s].at[0], isb0, si0)
            pltpu.async_copy(srcr.at[s].at[1], isb1, si1)
            plsc.subcore_barrier()

            pltpu.make_async_copy(srcr.at[s].at[0], isb0, si0).wait()
            pltpu.async_copy(y.at[q].at[isb0], rows0, sg0)
            pltpu.make_async_copy(srcr.at[s].at[1], isb1, si1).wait()
            pltpu.async_copy(y.at[q].at[isb1], rows1, sg1)

            def step2(t, carry):
                j0 = 2 * t
                more = t + 1 < T
                pltpu.make_async_copy(y.at[q].at[isb0], rows0, sg0).wait()
                pltpu.async_copy(rows0, aggsh.at[idx_d.at[j0]], ss0, add=True)

                @pl.when(more)
                def _():
                    pltpu.async_copy(srcr.at[s].at[j0 + 2], isb0, si0)
                pltpu.make_async_copy(y.at[q].at[isb1], rows1, sg1).wait()
                pltpu.async_copy(rows1, aggsh.at[idx_d.at[j0 + 1]], ss1,
                                 add=True)

                @pl.when(more)
                def _():
                    pltpu.async_copy(srcr.at[s].at[j0 + 3], isb1, si1)
                pltpu.make_async_copy(rows0, aggsh.at[idx_d.at[j0]],
                                      ss0).wait()

                @pl.when(more)
                def _():
                    pltpu.make_async_copy(srcr.at[s].at[j0 + 2], isb0,
                                          si0).wait()
                    pltpu.async_copy(y.at[q].at[isb0], rows0, sg0)
                pltpu.make_async_copy(rows1, aggsh.at[idx_d.at[j0 + 1]],
                                      ss1).wait()

                @pl.when(more)
                def _():
                    pltpu.make_async_copy(srcr.at[s].at[j0 + 3], isb1,
                                          si1).wait()
                    pltpu.async_copy(y.at[q].at[isb1], rows1, sg1)
                return carry
            lax.fori_loop(0, T, step2, 0)
            plsc.subcore_barrier()
            pltpu.sync_copy(aggsh.at[pl.ds(r0, RPT)],
                            out.at[q, pl.ds(r0, RPT)])

    return pl.kernel(body, out_type=out_type, mesh=mesh,
                     scratch_types=scratch)


def _make_sc_cnt():
    mesh = plsc.VectorSubcoreMesh(core_axis_name="c", subcore_axis_name="s",
                                  num_cores=2, num_subcores=NT)
    out_type = [jax.ShapeDtypeStruct((NPAD, 128), jnp.float32)]
    scratch = [
        pltpu.VMEM((STEPS, K), jnp.int32),
        pltpu.VMEM((K, 128), jnp.float32),
        pltpu.VMEM_SHARED((NPAD, 128), jnp.float32),
        pltpu.SemaphoreType.DMA,
    ]

    def body(dstr, z128, ones_h, cnt_out, idx_d, ones_v, cntsh, sem):
        c = lax.axis_index("c")
        s = lax.axis_index("s")
        r0 = s * RPT

        @pl.when(c == 0)
        def _():
            pltpu.sync_copy(dstr.at[s], idx_d)
            pltpu.sync_copy(ones_h, ones_v)
            pltpu.sync_copy(z128.at[pl.ds(r0, RPT)],
                            cntsh.at[pl.ds(r0, RPT)])
        plsc.subcore_barrier()

        @pl.when(c == 0)
        def _():
            def step(j, carry):
                pltpu.async_copy(ones_v, cntsh.at[idx_d.at[j]], sem, add=True)
                return carry
            lax.fori_loop(0, STEPS, step, 0)

            def drain(j, carry):
                pltpu.make_async_copy(ones_v, cntsh.at[idx_d.at[j]], sem).wait()
                return carry
            lax.fori_loop(0, STEPS, drain, 0)
        plsc.subcore_barrier()

        @pl.when(c == 0)
        def _():
            pltpu.sync_copy(cntsh.at[pl.ds(r0, RPT)],
                            cnt_out.at[pl.ds(r0, RPT)])

    return pl.kernel(body, out_type=out_type, mesh=mesh,
                     scratch_types=scratch)


_SC_AGG2 = _make_sc_agg(2)
_SC_AGG4 = _make_sc_agg(4)
_SC_CNT = _make_sc_cnt()


def _make_tc_layer(din, last):
    nin = din // 128

    def body(*refs):
        if last:
            (agg_ref, cnt_ref, h_ref, wl_ref, b_ref, wr_ref, bt_ref,
             h_out, g_out) = refs
        else:
            (agg_ref, cnt_ref, h_ref, wl_ref, b_ref, wr_ref, a_ref,
             h_out, hch_out) = refs
        aggf = jnp.concatenate([agg_ref[qq] for qq in range(nin)], axis=1)
        cnt = cnt_ref[:, 0:1]
        mean = aggf * (1.0 / jnp.maximum(cnt, 1.0))
        out = (jnp.dot(mean, wl_ref[...], preferred_element_type=jnp.float32)
               + b_ref[...]
               + jnp.dot(h_ref[...], wr_ref[...],
                         preferred_element_type=jnp.float32))
        if not last:
            a = a_ref[...]
            out = jnp.where(out >= 0.0, out, a * out)
            h_out[...] = out
            for qq in range(4):
                hch_out[qq] = out[:, qq * 128:(qq + 1) * 128]
        else:
            h_out[...] = out
            i = pl.program_id(0)

            @pl.when(i == 0)
            def _():
                g_out[...] = jnp.full((NG, D_H), -jnp.inf, jnp.float32)
            bt = bt_ref[...]
            loc = jnp.stack(
                [jnp.max(jnp.where(bt == gg, out, -jnp.inf), axis=0)
                 for gg in range(NG)], axis=0)
            g_out[...] = jnp.maximum(g_out[...], loc)

    in_specs = [
        pl.BlockSpec((nin, RB, 128), lambda i: (0, i, 0)),
        pl.BlockSpec((RB, 128), lambda i: (i, 0)),
        pl.BlockSpec((RB, din), lambda i: (i, 0)),
        pl.BlockSpec((din, D_H), lambda i: (0, 0)),
        pl.BlockSpec((1, D_H), lambda i: (0, 0)),
        pl.BlockSpec((din, D_H), lambda i: (0, 0)),
    ]
    if last:
        in_specs.append(pl.BlockSpec((RB, 1), lambda i: (i, 0)))
        out_specs = [
            pl.BlockSpec((RB, D_H), lambda i: (i, 0)),
            pl.BlockSpec((NG, D_H), lambda i: (0, 0)),
        ]
        out_shape = [
            jax.ShapeDtypeStruct((NPAD, D_H), jnp.float32),
            jax.ShapeDtypeStruct((NG, D_H), jnp.float32),
        ]
    else:
        in_specs.append(pl.BlockSpec((1, D_H), lambda i: (0, 0)))
        out_specs = [
            pl.BlockSpec((RB, D_H), lambda i: (i, 0)),
            pl.BlockSpec((4, RB, 128), lambda i: (0, i, 0)),
        ]
        out_shape = [
            jax.ShapeDtypeStruct((NPAD, D_H), jnp.float32),
            jax.ShapeDtypeStruct((4, NPAD, 128), jnp.float32),
        ]

    return pl.pallas_call(
        body,
        grid=(GRID,),
        in_specs=in_specs,
        out_specs=out_specs,
        out_shape=out_shape,
    )


_TC_L0 = _make_tc_layer(D_IN, last=False)
_TC_L1 = _make_tc_layer(D_H, last=False)
_TC_L2 = _make_tc_layer(D_H, last=True)


@jax.jit
def kernel(x, edge_index, batch, W_l0, b_l0, W_r0, W_l1, b_l1, W_r1,
           W_l2, b_l2, W_r2, prelu_a):
    src, dst = edge_index[0], edge_index[1]
    pad_e = EPAD - E
    src_r = jnp.concatenate(
        [src, jnp.zeros((pad_e,), jnp.int32)]).reshape(NT, STEPS, K)
    dst_r = jnp.concatenate(
        [dst, jnp.full((pad_e,), NPAD - 1, jnp.int32)]).reshape(NT, STEPS, K)

    x_pad = jnp.pad(x, ((0, NPAD - N), (0, 0)))
    x_ch = x_pad.reshape(NPAD, D_IN // 128, 128).transpose(1, 0, 2)
    z128 = jnp.zeros((NPAD, 128), jnp.float32)
    ones128 = jnp.ones((K, 128), jnp.float32)
    batch_p = jnp.pad(batch, (0, NPAD - N),
                      constant_values=NG).reshape(NPAD, 1)

    bl0 = b_l0.reshape(1, D_H)
    bl1 = b_l1.reshape(1, D_H)
    bl2 = b_l2.reshape(1, D_H)
    a_r = prelu_a.reshape(1, D_H)

    (cnt16,) = _SC_CNT(dst_r, z128, ones128)
    (agg0,) = _SC_AGG2(x_ch, src_r, dst_r, z128)
    h1, h1ch = _TC_L0(agg0, cnt16, x_pad, W_l0, bl0, W_r0, a_r)
    (agg1,) = _SC_AGG4(h1ch, src_r, dst_r, z128)
    h2, h2ch = _TC_L1(agg1, cnt16, h1, W_l1, bl1, W_r1, a_r)
    (agg2,) = _SC_AGG4(h2ch, src_r, dst_r, z128)
    h3, g = _TC_L2(agg2, cnt16, h2, W_l2, bl2, W_r2, batch_p)
    return h3[:N], g


# speedup vs baseline: 1.2246x; 1.2246x over previous
import functools

import jax
import jax.numpy as jnp
from jax import lax
from jax.experimental import pallas as pl
from jax.experimental.pallas import tpu as pltpu
from jax.experimental.pallas import tpu_sc as plsc

N = 10000
E = 160000
D_IN = 256
D_H = 512
NG = 16

NT = 16
NPAD = 10240
RPT = NPAD // NT
K = 128
STEPS = 80
EPT = K * STEPS
EPAD = NT * EPT
RB = 256
GRID = NPAD // RB


def _make_sc_agg(n_chunks):
    cps = n_chunks // 2
    mesh = plsc.VectorSubcoreMesh(core_axis_name="c", subcore_axis_name="s",
                                  num_cores=2, num_subcores=NT)
    out_type = [jax.ShapeDtypeStruct((n_chunks, NPAD, 128), jnp.float32)]
    scratch = [
        pltpu.VMEM((STEPS, K), jnp.int32),
        pltpu.VMEM((K,), jnp.int32),
        pltpu.VMEM((K,), jnp.int32),
        pltpu.VMEM((K,), jnp.int32),
        pltpu.VMEM((K, 128), jnp.float32),
        pltpu.VMEM((K, 128), jnp.float32),
        pltpu.VMEM_SHARED((NPAD, 128), jnp.float32),
        pltpu.SemaphoreType.DMA,
        pltpu.SemaphoreType.DMA,
    ]

    def body(y, pidxr, z128, out,
             pidx, sb0, sb1, db, rows0, rows1, aggsh, sg0, sg1):
        c = lax.axis_index("c")
        s = lax.axis_index("s")
        r0 = s * RPT
        T = STEPS // 2
        pltpu.sync_copy(pidxr.at[s], pidx)

        def unpack_src(j, buf):
            for i in range(K // 16):
                v = pidx[j, pl.ds(16 * i, 16)]
                buf[pl.ds(16 * i, 16)] = jnp.bitwise_and(v, 0xFFFF)

        def unpack_dst(j, buf):
            for i in range(K // 16):
                v = pidx[j, pl.ds(16 * i, 16)]
                buf[pl.ds(16 * i, 16)] = lax.shift_right_logical(v, 16)

        for qi in range(cps):
            q = c * cps + qi
            pltpu.sync_copy(z128.at[pl.ds(r0, RPT)], aggsh.at[pl.ds(r0, RPT)])
            plsc.subcore_barrier()

            unpack_src(0, sb0)
            pltpu.async_copy(y.at[q].at[sb0], rows0, sg0)
            unpack_src(1, sb1)
            pltpu.async_copy(y.at[q].at[sb1], rows1, sg1)

            def step2(t, carry):
                j0 = 2 * t
                more = t + 1 < T
                unpack_dst(j0, db)
                pltpu.make_async_copy(y.at[q].at[sb0], rows0, sg0).wait()
                pltpu.sync_copy(rows0, aggsh.at[db], add=True)

                @pl.when(more)
                def _():
                    unpack_src(j0 + 2, sb0)
                    pltpu.async_copy(y.at[q].at[sb0], rows0, sg0)
                unpack_dst(j0 + 1, db)
                pltpu.make_async_copy(y.at[q].at[sb1], rows1, sg1).wait()
                pltpu.sync_copy(rows1, aggsh.at[db], add=True)

                @pl.when(more)
                def _():
                    unpack_src(j0 + 3, sb1)
                    pltpu.async_copy(y.at[q].at[sb1], rows1, sg1)
                return carry
            lax.fori_loop(0, T, step2, 0)
            plsc.subcore_barrier()
            pltpu.sync_copy(aggsh.at[pl.ds(r0, RPT)],
                            out.at[q, pl.ds(r0, RPT)])

    return pl.kernel(body, out_type=out_type, mesh=mesh,
                     scratch_types=scratch)


def _make_sc_cnt():
    mesh = plsc.VectorSubcoreMesh(core_axis_name="c", subcore_axis_name="s",
                                  num_cores=2, num_subcores=NT)
    out_type = [jax.ShapeDtypeStruct((NPAD, 128), jnp.float32)]
    scratch = [
        pltpu.VMEM((STEPS, K), jnp.int32),
        pltpu.VMEM((K, 128), jnp.float32),
        pltpu.VMEM_SHARED((NPAD, 128), jnp.float32),
        pltpu.SemaphoreType.DMA,
    ]

    def body(dstr, z128, ones_h, cnt_out, idx_d, ones_v, cntsh, sem):
        c = lax.axis_index("c")
        s = lax.axis_index("s")
        r0 = s * RPT

        @pl.when(c == 0)
        def _():
            pltpu.sync_copy(dstr.at[s], idx_d)
            pltpu.sync_copy(ones_h, ones_v)
            pltpu.sync_copy(z128.at[pl.ds(r0, RPT)],
                            cntsh.at[pl.ds(r0, RPT)])
        plsc.subcore_barrier()

        @pl.when(c == 0)
        def _():
            def step(j, carry):
                pltpu.async_copy(ones_v, cntsh.at[idx_d.at[j]], sem, add=True)
                return carry
            lax.fori_loop(0, STEPS, step, 0)

            def drain(j, carry):
                pltpu.make_async_copy(ones_v, cntsh.at[idx_d.at[j]], sem).wait()
                return carry
            lax.fori_loop(0, STEPS, drain, 0)
        plsc.subcore_barrier()

        @pl.when(c == 0)
        def _():
            pltpu.sync_copy(cntsh.at[pl.ds(r0, RPT)],
                            cnt_out.at[pl.ds(r0, RPT)])

    return pl.kernel(body, out_type=out_type, mesh=mesh,
                     scratch_types=scratch)


_SC_AGG2 = _make_sc_agg(2)
_SC_AGG4 = _make_sc_agg(4)
_SC_CNT = _make_sc_cnt()


def _make_tc_layer(din, last):
    nin = din // 128

    def body(*refs):
        if last:
            (agg_ref, cnt_ref, h_ref, wl_ref, b_ref, wr_ref, bt_ref,
             h_out, g_out) = refs
        else:
            (agg_ref, cnt_ref, h_ref, wl_ref, b_ref, wr_ref, a_ref,
             h_out, hch_out) = refs
        aggf = jnp.concatenate([agg_ref[qq] for qq in range(nin)], axis=1)
        cnt = cnt_ref[:, 0:1]
        mean = aggf * (1.0 / jnp.maximum(cnt, 1.0))
        out = (jnp.dot(mean, wl_ref[...], preferred_element_type=jnp.float32)
               + b_ref[...]
               + jnp.dot(h_ref[...], wr_ref[...],
                         preferred_element_type=jnp.float32))
        if not last:
            a = a_ref[...]
            out = jnp.where(out >= 0.0, out, a * out)
            h_out[...] = out
            for qq in range(4):
                hch_out[qq] = out[:, qq * 128:(qq + 1) * 128]
        else:
            h_out[...] = out
            i = pl.program_id(0)

            @pl.when(i == 0)
            def _():
                g_out[...] = jnp.full((NG, D_H), -jnp.inf, jnp.float32)
            bt = bt_ref[...]
            loc = jnp.stack(
                [jnp.max(jnp.where(bt == gg, out, -jnp.inf), axis=0)
                 for gg in range(NG)], axis=0)
            g_out[...] = jnp.maximum(g_out[...], loc)

    in_specs = [
        pl.BlockSpec((nin, RB, 128), lambda i: (0, i, 0)),
        pl.BlockSpec((RB, 128), lambda i: (i, 0)),
        pl.BlockSpec((RB, din), lambda i: (i, 0)),
        pl.BlockSpec((din, D_H), lambda i: (0, 0)),
        pl.BlockSpec((1, D_H), lambda i: (0, 0)),
        pl.BlockSpec((din, D_H), lambda i: (0, 0)),
    ]
    if last:
        in_specs.append(pl.BlockSpec((RB, 1), lambda i: (i, 0)))
        out_specs = [
            pl.BlockSpec((RB, D_H), lambda i: (i, 0)),
            pl.BlockSpec((NG, D_H), lambda i: (0, 0)),
        ]
        out_shape = [
            jax.ShapeDtypeStruct((NPAD, D_H), jnp.float32),
            jax.ShapeDtypeStruct((NG, D_H), jnp.float32),
        ]
    else:
        in_specs.append(pl.BlockSpec((1, D_H), lambda i: (0, 0)))
        out_specs = [
            pl.BlockSpec((RB, D_H), lambda i: (i, 0)),
            pl.BlockSpec((4, RB, 128), lambda i: (0, i, 0)),
        ]
        out_shape = [
            jax.ShapeDtypeStruct((NPAD, D_H), jnp.float32),
            jax.ShapeDtypeStruct((4, NPAD, 128), jnp.float32),
        ]

    return pl.pallas_call(
        body,
        grid=(GRID,),
        in_specs=in_specs,
        out_specs=out_specs,
        out_shape=out_shape,
    )


_TC_L0 = _make_tc_layer(D_IN, last=False)
_TC_L1 = _make_tc_layer(D_H, last=False)
_TC_L2 = _make_tc_layer(D_H, last=True)


@jax.jit
def kernel(x, edge_index, batch, W_l0, b_l0, W_r0, W_l1, b_l1, W_r1,
           W_l2, b_l2, W_r2, prelu_a):
    src, dst = edge_index[0], edge_index[1]
    pad_e = EPAD - E
    src_p = jnp.concatenate([src, jnp.zeros((pad_e,), jnp.int32)])
    dst_p = jnp.concatenate([dst, jnp.full((pad_e,), NPAD - 1, jnp.int32)])
    pidx_r = (src_p | (dst_p << 16)).reshape(NT, STEPS, K)
    dst_r = dst_p.reshape(NT, STEPS, K)

    x_pad = jnp.pad(x, ((0, NPAD - N), (0, 0)))
    x_ch = x_pad.reshape(NPAD, D_IN // 128, 128).transpose(1, 0, 2)
    z128 = jnp.zeros((NPAD, 128), jnp.float32)
    ones128 = jnp.ones((K, 128), jnp.float32)
    batch_p = jnp.pad(batch, (0, NPAD - N),
                      constant_values=NG).reshape(NPAD, 1)

    bl0 = b_l0.reshape(1, D_H)
    bl1 = b_l1.reshape(1, D_H)
    bl2 = b_l2.reshape(1, D_H)
    a_r = prelu_a.reshape(1, D_H)

    (cnt16,) = _SC_CNT(dst_r, z128, ones128)
    (agg0,) = _SC_AGG2(x_ch, pidx_r, z128)
    h1, h1ch = _TC_L0(agg0, cnt16, x_pad, W_l0, bl0, W_r0, a_r)
    (agg1,) = _SC_AGG4(h1ch, pidx_r, z128)
    h2, h2ch = _TC_L1(agg1, cnt16, h1, W_l1, bl1, W_r1, a_r)
    (agg2,) = _SC_AGG4(h2ch, pidx_r, z128)
    h3, g = _TC_L2(agg2, cnt16, h2, W_l2, bl2, W_r2, batch_p)
    return h3[:N], g


# speedup vs baseline: 1.4682x; 1.1989x over previous
import jax
import jax.numpy as jnp
from jax import lax
from jax.experimental import pallas as pl
from jax.experimental.pallas import tpu as pltpu
from jax.experimental.pallas import tpu_sc as plsc

N = 10000
E = 160000
D_IN = 256
D_H = 512
NG = 16

NT = 16
NPAD = 10240
RPT = NPAD // NT
K = 128
STEPS = 80
EPT = K * STEPS
EPAD = NT * EPT
RB = 256
GRID = NPAD // RB


def _sc_agg_body(y, pidxr, zinit, out, pidx, sb0, sb1, db, rows0, rows1,
                 aggsh, sg0, sg1):
    c = lax.axis_index("c")
    s = lax.axis_index("s")
    r0 = s * RPT
    T = STEPS // 2
    pltpu.sync_copy(pidxr.at[s], pidx)

    def unpack_src(j, buf):
        for i in range(K // 16):
            v = pidx[j, pl.ds(16 * i, 16)]
            buf[pl.ds(16 * i, 16)] = jnp.bitwise_and(v, 0xFFFF)

    def unpack_dst(j, buf):
        for i in range(K // 16):
            v = pidx[j, pl.ds(16 * i, 16)]
            buf[pl.ds(16 * i, 16)] = lax.shift_right_logical(v, 16)

    q = c
    pltpu.sync_copy(zinit.at[pl.ds(r0, RPT)], aggsh.at[pl.ds(r0, RPT)])
    plsc.subcore_barrier()

    unpack_src(0, sb0)
    pltpu.async_copy(y.at[q].at[sb0], rows0, sg0)
    unpack_src(1, sb1)
    pltpu.async_copy(y.at[q].at[sb1], rows1, sg1)

    def step2(t, carry):
        j0 = 2 * t
        more = t + 1 < T
        unpack_dst(j0, db)
        pltpu.make_async_copy(y.at[q].at[sb0], rows0, sg0).wait()
        pltpu.sync_copy(rows0, aggsh.at[db], add=True)

        @pl.when(more)
        def _():
            unpack_src(j0 + 2, sb0)
            pltpu.async_copy(y.at[q].at[sb0], rows0, sg0)
        unpack_dst(j0 + 1, db)
        pltpu.make_async_copy(y.at[q].at[sb1], rows1, sg1).wait()
        pltpu.sync_copy(rows1, aggsh.at[db], add=True)

        @pl.when(more)
        def _():
            unpack_src(j0 + 3, sb1)
            pltpu.async_copy(y.at[q].at[sb1], rows1, sg1)
        return carry
    lax.fori_loop(0, T, step2, 0)
    plsc.subcore_barrier()
    pltpu.sync_copy(aggsh.at[pl.ds(r0, RPT)], out.at[q, pl.ds(r0, RPT)])


def _make_sc_agg_f32():
    mesh = plsc.VectorSubcoreMesh(core_axis_name="c", subcore_axis_name="s",
                                  num_cores=2, num_subcores=NT)
    out_type = [jax.ShapeDtypeStruct((2, NPAD, 128), jnp.float32)]
    scratch = [
        pltpu.VMEM((STEPS, K), jnp.int32),
        pltpu.VMEM((K,), jnp.int32),
        pltpu.VMEM((K,), jnp.int32),
        pltpu.VMEM((K,), jnp.int32),
        pltpu.VMEM((K, 128), jnp.float32),
        pltpu.VMEM((K, 128), jnp.float32),
        pltpu.VMEM_SHARED((NPAD, 128), jnp.float32),
        pltpu.SemaphoreType.DMA,
        pltpu.SemaphoreType.DMA,
    ]

    def body(y, pidxr, zinit, out, pidx, sb0, sb1, db, rows0, rows1,
             aggsh, sg0, sg1):
        _sc_agg_body(y, pidxr, zinit, out, pidx, sb0, sb1, db, rows0, rows1,
                     aggsh, sg0, sg1)

    return pl.kernel(body, out_type=out_type, mesh=mesh,
                     scratch_types=scratch)


def _make_sc_agg_bf16():
    mesh = plsc.VectorSubcoreMesh(core_axis_name="c", subcore_axis_name="s",
                                  num_cores=2, num_subcores=NT)
    out_type = [jax.ShapeDtypeStruct((2, NPAD, 2, 128), jnp.bfloat16)]
    scratch = [
        pltpu.VMEM((STEPS, K), jnp.int32),
        pltpu.VMEM((K,), jnp.int32),
        pltpu.VMEM((K,), jnp.int32),
        pltpu.VMEM((K,), jnp.int32),
        pltpu.VMEM((K, 2, 128), jnp.bfloat16),
        pltpu.VMEM((K, 2, 128), jnp.bfloat16),
        pltpu.VMEM_SHARED((NPAD, 2, 128), jnp.bfloat16),
        pltpu.SemaphoreType.DMA,
        pltpu.SemaphoreType.DMA,
    ]

    def body(y, pidxr, zinit, out, pidx, sb0, sb1, db, rows0, rows1,
             aggsh, sg0, sg1):
        _sc_agg_body(y, pidxr, zinit, out, pidx, sb0, sb1, db, rows0, rows1,
                     aggsh, sg0, sg1)

    return pl.kernel(body, out_type=out_type, mesh=mesh,
                     scratch_types=scratch,
                     compiler_params=pltpu.CompilerParams(
                         use_tc_tiling_on_sc=False))


def _make_sc_cnt():
    mesh = plsc.VectorSubcoreMesh(core_axis_name="c", subcore_axis_name="s",
                                  num_cores=2, num_subcores=NT)
    out_type = [jax.ShapeDtypeStruct((NPAD, 128), jnp.float32)]
    scratch = [
        pltpu.VMEM((STEPS, K), jnp.int32),
        pltpu.VMEM((K, 128), jnp.float32),
        pltpu.VMEM_SHARED((NPAD, 128), jnp.float32),
        pltpu.SemaphoreType.DMA,
    ]

    def body(dstr, z128, ones_h, cnt_out, idx_d, ones_v, cntsh, sem):
        c = lax.axis_index("c")
        s = lax.axis_index("s")
        r0 = s * RPT

        @pl.when(c == 0)
        def _():
            pltpu.sync_copy(dstr.at[s], idx_d)
            pltpu.sync_copy(ones_h, ones_v)
            pltpu.sync_copy(z128.at[pl.ds(r0, RPT)],
                            cntsh.at[pl.ds(r0, RPT)])
        plsc.subcore_barrier()

        @pl.when(c == 0)
        def _():
            def step(j, carry):
                pltpu.async_copy(ones_v, cntsh.at[idx_d.at[j]], sem, add=True)
                return carry
            lax.fori_loop(0, STEPS, step, 0)

            def drain(j, carry):
                pltpu.make_async_copy(ones_v, cntsh.at[idx_d.at[j]],
                                      sem).wait()
                return carry
            lax.fori_loop(0, STEPS, drain, 0)
        plsc.subcore_barrier()

        @pl.when(c == 0)
        def _():
            pltpu.sync_copy(cntsh.at[pl.ds(r0, RPT)],
                            cnt_out.at[pl.ds(r0, RPT)])

    return pl.kernel(body, out_type=out_type, mesh=mesh,
                     scratch_types=scratch)


_SC_AGG_F32 = _make_sc_agg_f32()
_SC_AGG_BF16 = _make_sc_agg_bf16()
_SC_CNT = _make_sc_cnt()


def _make_tc_layer(din, last, agg_bf16):
    nin = din // 128

    def body(*refs):
        if last:
            (agg_ref, cnt_ref, h_ref, wl_ref, b_ref, wr_ref, bt_ref,
             h_out, g_out) = refs
        else:
            (agg_ref, cnt_ref, h_ref, wl_ref, b_ref, wr_ref, a_ref,
             h_out, hch_out) = refs
        if agg_bf16:
            aggf = jnp.concatenate(
                [agg_ref[qq].reshape(RB, din // 2) for qq in range(2)],
                axis=1).astype(jnp.float32)
        else:
            aggf = jnp.concatenate([agg_ref[qq] for qq in range(nin)],
                                   axis=1)
        cnt = cnt_ref[:, 0:1]
        mean = aggf * (1.0 / jnp.maximum(cnt, 1.0))
        out = (jnp.dot(mean, wl_ref[...], preferred_element_type=jnp.float32)
               + b_ref[...]
               + jnp.dot(h_ref[...], wr_ref[...],
                         preferred_element_type=jnp.float32))
        if not last:
            a = a_ref[...]
            out = jnp.where(out >= 0.0, out, a * out)
            h_out[...] = out
            for qq in range(2):
                hch_out[qq] = out[:, qq * 256:(qq + 1) * 256].reshape(
                    RB, 2, 128).astype(jnp.bfloat16)
        else:
            h_out[...] = out
            i = pl.program_id(0)

            @pl.when(i == 0)
            def _():
                g_out[...] = jnp.full((NG, D_H), -jnp.inf, jnp.float32)
            bt = bt_ref[...]
            loc = jnp.stack(
                [jnp.max(jnp.where(bt == gg, out, -jnp.inf), axis=0)
                 for gg in range(NG)], axis=0)
            g_out[...] = jnp.maximum(g_out[...], loc)

    if agg_bf16:
        agg_spec = pl.BlockSpec((2, RB, 2, 128), lambda i: (0, i, 0, 0))
    else:
        agg_spec = pl.BlockSpec((nin, RB, 128), lambda i: (0, i, 0))
    in_specs = [
        agg_spec,
        pl.BlockSpec((RB, 128), lambda i: (i, 0)),
        pl.BlockSpec((RB, din), lambda i: (i, 0)),
        pl.BlockSpec((din, D_H), lambda i: (0, 0)),
        pl.BlockSpec((1, D_H), lambda i: (0, 0)),
        pl.BlockSpec((din, D_H), lambda i: (0, 0)),
    ]
    if last:
        in_specs.append(pl.BlockSpec((RB, 1), lambda i: (i, 0)))
        out_specs = [
            pl.BlockSpec((RB, D_H), lambda i: (i, 0)),
            pl.BlockSpec((NG, D_H), lambda i: (0, 0)),
        ]
        out_shape = [
            jax.ShapeDtypeStruct((NPAD, D_H), jnp.float32),
            jax.ShapeDtypeStruct((NG, D_H), jnp.float32),
        ]
    else:
        in_specs.append(pl.BlockSpec((1, D_H), lambda i: (0, 0)))
        out_specs = [
            pl.BlockSpec((RB, D_H), lambda i: (i, 0)),
            pl.BlockSpec((2, RB, 2, 128), lambda i: (0, i, 0, 0)),
        ]
        out_shape = [
            jax.ShapeDtypeStruct((NPAD, D_H), jnp.float32),
            jax.ShapeDtypeStruct((2, NPAD, 2, 128), jnp.bfloat16),
        ]

    return pl.pallas_call(
        body,
        grid=(GRID,),
        in_specs=in_specs,
        out_specs=out_specs,
        out_shape=out_shape,
    )


_TC_L0 = _make_tc_layer(D_IN, last=False, agg_bf16=False)
_TC_L1 = _make_tc_layer(D_H, last=False, agg_bf16=True)
_TC_L2 = _make_tc_layer(D_H, last=True, agg_bf16=True)


@jax.jit
def kernel(x, edge_index, batch, W_l0, b_l0, W_r0, W_l1, b_l1, W_r1,
           W_l2, b_l2, W_r2, prelu_a):
    src, dst = edge_index[0], edge_index[1]
    pad_e = EPAD - E
    src_p = jnp.concatenate([src, jnp.zeros((pad_e,), jnp.int32)])
    dst_p = jnp.concatenate([dst, jnp.full((pad_e,), NPAD - 1, jnp.int32)])
    pidx_r = (src_p | (dst_p << 16)).reshape(NT, STEPS, K)
    dst_r = dst_p.reshape(NT, STEPS, K)

    x_pad = jnp.pad(x, ((0, NPAD - N), (0, 0)))
    x_ch = x_pad.reshape(NPAD, D_IN // 128, 128).transpose(1, 0, 2)
    z128 = jnp.zeros((NPAD, 128), jnp.float32)
    zb = jnp.zeros((NPAD, 2, 128), jnp.bfloat16)
    ones128 = jnp.ones((K, 128), jnp.float32)
    batch_p = jnp.pad(batch, (0, NPAD - N),
                      constant_values=NG).reshape(NPAD, 1)

    bl0 = b_l0.reshape(1, D_H)
    bl1 = b_l1.reshape(1, D_H)
    bl2 = b_l2.reshape(1, D_H)
    a_r = prelu_a.reshape(1, D_H)

    (cnt16,) = _SC_CNT(dst_r, z128, ones128)
    (agg0,) = _SC_AGG_F32(x_ch, pidx_r, z128)
    h1, h1ch = _TC_L0(agg0, cnt16, x_pad, W_l0, bl0, W_r0, a_r)
    (agg1,) = _SC_AGG_BF16(h1ch, pidx_r, zb)
    h2, h2ch = _TC_L1(agg1, cnt16, h1, W_l1, bl1, W_r1, a_r)
    (agg2,) = _SC_AGG_BF16(h2ch, pidx_r, zb)
    h3, g = _TC_L2(agg2, cnt16, h2, W_l2, bl2, W_r2, batch_p)
    return h3[:N], g


# speedup vs baseline: 1.5011x; 1.0224x over previous
import jax
import jax.numpy as jnp
from jax import lax
from jax.experimental import pallas as pl
from jax.experimental.pallas import tpu as pltpu
from jax.experimental.pallas import tpu_sc as plsc

N = 10000
E = 160000
D_IN = 256
D_H = 512
NG = 16

NT = 16
NPAD = 10240
RPT = NPAD // NT
K = 128
STEPS = 80
EPT = K * STEPS
EPAD = NT * EPT
RB = 256
GRID = NPAD // RB


def _sc_agg_body(y, pidxr, zinit, out, pidx, sb0, sb1, db, rows0, rows1,
                 aggsh, sg0, sg1):
    c = lax.axis_index("c")
    s = lax.axis_index("s")
    r0 = s * RPT
    T = STEPS // 2
    pltpu.sync_copy(pidxr.at[s], pidx)

    def unpack_src(j, buf):
        for i in range(K // 16):
            v = pidx[j, pl.ds(16 * i, 16)]
            buf[pl.ds(16 * i, 16)] = jnp.bitwise_and(v, 0xFFFF)

    def unpack_dst(j, buf):
        for i in range(K // 16):
            v = pidx[j, pl.ds(16 * i, 16)]
            buf[pl.ds(16 * i, 16)] = lax.shift_right_logical(v, 16)

    q = c
    pltpu.sync_copy(zinit.at[pl.ds(r0, RPT)], aggsh.at[pl.ds(r0, RPT)])
    plsc.subcore_barrier()

    unpack_src(0, sb0)
    pltpu.async_copy(y.at[q].at[sb0], rows0, sg0)
    unpack_src(1, sb1)
    pltpu.async_copy(y.at[q].at[sb1], rows1, sg1)

    def step2(t, carry):
        j0 = 2 * t
        more = t + 1 < T
        unpack_dst(j0, db)
        pltpu.make_async_copy(y.at[q].at[sb0], rows0, sg0).wait()
        pltpu.sync_copy(rows0, aggsh.at[db], add=True)

        @pl.when(more)
        def _():
            unpack_src(j0 + 2, sb0)
            pltpu.async_copy(y.at[q].at[sb0], rows0, sg0)
        unpack_dst(j0 + 1, db)
        pltpu.make_async_copy(y.at[q].at[sb1], rows1, sg1).wait()
        pltpu.sync_copy(rows1, aggsh.at[db], add=True)

        @pl.when(more)
        def _():
            unpack_src(j0 + 3, sb1)
            pltpu.async_copy(y.at[q].at[sb1], rows1, sg1)
        return carry
    lax.fori_loop(0, T, step2, 0)
    plsc.subcore_barrier()
    pltpu.sync_copy(aggsh.at[pl.ds(r0, RPT)], out.at[q, pl.ds(r0, RPT)])


S2 = STEPS // 2


def _make_sc_agg0():
    mesh = plsc.VectorSubcoreMesh(core_axis_name="c", subcore_axis_name="s",
                                  num_cores=2, num_subcores=NT)
    out_type = [jax.ShapeDtypeStruct((2, NPAD, 2, 128), jnp.bfloat16)]
    scratch = [
        pltpu.VMEM((S2, K), jnp.int32),
        pltpu.VMEM((K,), jnp.int32),
        pltpu.VMEM((K,), jnp.int32),
        pltpu.VMEM((K,), jnp.int32),
        pltpu.VMEM((K, 2, 128), jnp.bfloat16),
        pltpu.VMEM((K, 2, 128), jnp.bfloat16),
        pltpu.VMEM_SHARED((NPAD, 2, 128), jnp.bfloat16),
        pltpu.SemaphoreType.DMA,
        pltpu.SemaphoreType.DMA,
    ]

    def body(y, pidxr, zinit, out, pidx, sb0, sb1, db, rows0, rows1,
             aggsh, sg0, sg1):
        c = lax.axis_index("c")
        s = lax.axis_index("s")
        r0 = s * RPT
        T = S2 // 2
        pltpu.sync_copy(pidxr.at[c * NT + s], pidx)

        def unpack_src(j, buf):
            for i in range(K // 16):
                v = pidx[j, pl.ds(16 * i, 16)]
                buf[pl.ds(16 * i, 16)] = jnp.bitwise_and(v, 0xFFFF)

        def unpack_dst(j, buf):
            for i in range(K // 16):
                v = pidx[j, pl.ds(16 * i, 16)]
                buf[pl.ds(16 * i, 16)] = lax.shift_right_logical(v, 16)

        pltpu.sync_copy(zinit.at[pl.ds(r0, RPT)], aggsh.at[pl.ds(r0, RPT)])
        plsc.subcore_barrier()

        unpack_src(0, sb0)
        pltpu.async_copy(y.at[sb0], rows0, sg0)
        unpack_src(1, sb1)
        pltpu.async_copy(y.at[sb1], rows1, sg1)

        def step2(t, carry):
            j0 = 2 * t
            more = t + 1 < T
            unpack_dst(j0, db)
            pltpu.make_async_copy(y.at[sb0], rows0, sg0).wait()
            pltpu.sync_copy(rows0, aggsh.at[db], add=True)

            @pl.when(more)
            def _():
                unpack_src(j0 + 2, sb0)
                pltpu.async_copy(y.at[sb0], rows0, sg0)
            unpack_dst(j0 + 1, db)
            pltpu.make_async_copy(y.at[sb1], rows1, sg1).wait()
            pltpu.sync_copy(rows1, aggsh.at[db], add=True)

            @pl.when(more)
            def _():
                unpack_src(j0 + 3, sb1)
                pltpu.async_copy(y.at[sb1], rows1, sg1)
            return carry
        lax.fori_loop(0, T, step2, 0)
        plsc.subcore_barrier()
        pltpu.sync_copy(aggsh.at[pl.ds(r0, RPT)], out.at[c, pl.ds(r0, RPT)])

    return pl.kernel(body, out_type=out_type, mesh=mesh,
                     scratch_types=scratch,
                     compiler_params=pltpu.CompilerParams(
                         use_tc_tiling_on_sc=False))


def _make_sc_agg_bf16():
    mesh = plsc.VectorSubcoreMesh(core_axis_name="c", subcore_axis_name="s",
                                  num_cores=2, num_subcores=NT)
    out_type = [jax.ShapeDtypeStruct((2, NPAD, 2, 128), jnp.bfloat16)]
    scratch = [
        pltpu.VMEM((STEPS, K), jnp.int32),
        pltpu.VMEM((K,), jnp.int32),
        pltpu.VMEM((K,), jnp.int32),
        pltpu.VMEM((K,), jnp.int32),
        pltpu.VMEM((K, 2, 128), jnp.bfloat16),
        pltpu.VMEM((K, 2, 128), jnp.bfloat16),
        pltpu.VMEM_SHARED((NPAD, 2, 128), jnp.bfloat16),
        pltpu.SemaphoreType.DMA,
        pltpu.SemaphoreType.DMA,
    ]

    def body(y, pidxr, zinit, out, pidx, sb0, sb1, db, rows0, rows1,
             aggsh, sg0, sg1):
        _sc_agg_body(y, pidxr, zinit, out, pidx, sb0, sb1, db, rows0, rows1,
                     aggsh, sg0, sg1)

    return pl.kernel(body, out_type=out_type, mesh=mesh,
                     scratch_types=scratch,
                     compiler_params=pltpu.CompilerParams(
                         use_tc_tiling_on_sc=False))


def _make_sc_cnt():
    mesh = plsc.VectorSubcoreMesh(core_axis_name="c", subcore_axis_name="s",
                                  num_cores=2, num_subcores=NT)
    out_type = [jax.ShapeDtypeStruct((2, NPAD, 128), jnp.float32)]
    scratch = [
        pltpu.VMEM((S2, K), jnp.int32),
        pltpu.VMEM((K, 128), jnp.float32),
        pltpu.VMEM_SHARED((NPAD, 128), jnp.float32),
        pltpu.SemaphoreType.DMA,
    ]

    def body(dstr, z128, ones_h, cnt_out, idx_d, ones_v, cntsh, sem):
        c = lax.axis_index("c")
        s = lax.axis_index("s")
        r0 = s * RPT
        pltpu.sync_copy(dstr.at[c * NT + s], idx_d)
        pltpu.sync_copy(ones_h, ones_v)
        pltpu.sync_copy(z128.at[pl.ds(r0, RPT)], cntsh.at[pl.ds(r0, RPT)])
        plsc.subcore_barrier()

        def step(j, carry):
            pltpu.async_copy(ones_v, cntsh.at[idx_d.at[j]], sem, add=True)
            return carry
        lax.fori_loop(0, S2, step, 0)

        def drain(j, carry):
            pltpu.make_async_copy(ones_v, cntsh.at[idx_d.at[j]], sem).wait()
            return carry
        lax.fori_loop(0, S2, drain, 0)
        plsc.subcore_barrier()
        pltpu.sync_copy(cntsh.at[pl.ds(r0, RPT)], cnt_out.at[c, pl.ds(r0, RPT)])

    return pl.kernel(body, out_type=out_type, mesh=mesh,
                     scratch_types=scratch)


_SC_AGG0 = _make_sc_agg0()
_SC_AGG_BF16 = _make_sc_agg_bf16()
_SC_CNT = _make_sc_cnt()


def _make_tc_layer(din, last, agg_bf16):
    nin = din // 128

    def body(*refs):
        if last:
            (agg_ref, cnt_ref, h_ref, wl_ref, b_ref, wr_ref, bt_ref,
             h_out, g_out) = refs
        else:
            (agg_ref, cnt_ref, h_ref, wl_ref, b_ref, wr_ref, a_ref,
             h_out, hch_out) = refs
        if agg_bf16:
            aggf = jnp.concatenate(
                [agg_ref[qq].reshape(RB, din // 2) for qq in range(2)],
                axis=1).astype(jnp.float32)
        else:
            aggf = (agg_ref[0].astype(jnp.float32)
                    + agg_ref[1].astype(jnp.float32)).reshape(RB, din)
        cnt = cnt_ref[0, :, 0:1] + cnt_ref[1, :, 0:1]
        mean = aggf * (1.0 / jnp.maximum(cnt, 1.0))
        out = (jnp.dot(mean, wl_ref[...], preferred_element_type=jnp.float32)
               + b_ref[...]
               + jnp.dot(h_ref[...], wr_ref[...],
                         preferred_element_type=jnp.float32))
        if not last:
            a = a_ref[...]
            out = jnp.where(out >= 0.0, out, a * out)
            h_out[...] = out
            for qq in range(2):
                hch_out[qq] = out[:, qq * 256:(qq + 1) * 256].reshape(
                    RB, 2, 128).astype(jnp.bfloat16)
        else:
            h_out[...] = out
            i = pl.program_id(0)

            @pl.when(i == 0)
            def _():
                g_out[...] = jnp.full((NG, D_H), -jnp.inf, jnp.float32)
            bt = bt_ref[...]
            loc = jnp.stack(
                [jnp.max(jnp.where(bt == gg, out, -jnp.inf), axis=0)
                 for gg in range(NG)], axis=0)
            g_out[...] = jnp.maximum(g_out[...], loc)

    agg_spec = pl.BlockSpec((2, RB, 2, 128), lambda i: (0, i, 0, 0))
    in_specs = [
        agg_spec,
        pl.BlockSpec((2, RB, 128), lambda i: (0, i, 0)),
        pl.BlockSpec((RB, din), lambda i: (i, 0)),
        pl.BlockSpec((din, D_H), lambda i: (0, 0)),
        pl.BlockSpec((1, D_H), lambda i: (0, 0)),
        pl.BlockSpec((din, D_H), lambda i: (0, 0)),
    ]
    if last:
        in_specs.append(pl.BlockSpec((RB, 1), lambda i: (i, 0)))
        out_specs = [
            pl.BlockSpec((RB, D_H), lambda i: (i, 0)),
            pl.BlockSpec((NG, D_H), lambda i: (0, 0)),
        ]
        out_shape = [
            jax.ShapeDtypeStruct((NPAD, D_H), jnp.float32),
            jax.ShapeDtypeStruct((NG, D_H), jnp.float32),
        ]
    else:
        in_specs.append(pl.BlockSpec((1, D_H), lambda i: (0, 0)))
        out_specs = [
            pl.BlockSpec((RB, D_H), lambda i: (i, 0)),
            pl.BlockSpec((2, RB, 2, 128), lambda i: (0, i, 0, 0)),
        ]
        out_shape = [
            jax.ShapeDtypeStruct((NPAD, D_H), jnp.float32),
            jax.ShapeDtypeStruct((2, NPAD, 2, 128), jnp.bfloat16),
        ]

    return pl.pallas_call(
        body,
        grid=(GRID,),
        in_specs=in_specs,
        out_specs=out_specs,
        out_shape=out_shape,
    )


_TC_L0 = _make_tc_layer(D_IN, last=False, agg_bf16=False)
_TC_L1 = _make_tc_layer(D_H, last=False, agg_bf16=True)
_TC_L2 = _make_tc_layer(D_H, last=True, agg_bf16=True)


@jax.jit
def kernel(x, edge_index, batch, W_l0, b_l0, W_r0, W_l1, b_l1, W_r1,
           W_l2, b_l2, W_r2, prelu_a):
    src, dst = edge_index[0], edge_index[1]
    pad_e = EPAD - E
    src_p = jnp.concatenate([src, jnp.zeros((pad_e,), jnp.int32)])
    dst_p = jnp.concatenate([dst, jnp.full((pad_e,), NPAD - 1, jnp.int32)])
    pidx_p = src_p | (dst_p << 16)
    pidx_r = pidx_p.reshape(NT, STEPS, K)
    pidx_sp = pidx_p.reshape(2 * NT, S2, K)
    dst_r = dst_p.reshape(2 * NT, S2, K)

    x_pad = jnp.pad(x, ((0, NPAD - N), (0, 0)))
    x_ch = x_pad.reshape(NPAD, 2, 128).astype(jnp.bfloat16)
    z128 = jnp.zeros((NPAD, 128), jnp.float32)
    zb = jnp.zeros((NPAD, 2, 128), jnp.bfloat16)
    ones128 = jnp.ones((K, 128), jnp.float32)
    batch_p = jnp.pad(batch, (0, NPAD - N),
                      constant_values=NG).reshape(NPAD, 1)

    bl0 = b_l0.reshape(1, D_H)
    bl1 = b_l1.reshape(1, D_H)
    bl2 = b_l2.reshape(1, D_H)
    a_r = prelu_a.reshape(1, D_H)

    (cnt16,) = _SC_CNT(dst_r, z128, ones128)
    (agg0,) = _SC_AGG0(x_ch, pidx_sp, zb)
    h1, h1ch = _TC_L0(agg0, cnt16, x_pad, W_l0, bl0, W_r0, a_r)
    (agg1,) = _SC_AGG_BF16(h1ch, pidx_r, zb)
    h2, h2ch = _TC_L1(agg1, cnt16, h1, W_l1, bl1, W_r1, a_r)
    (agg2,) = _SC_AGG_BF16(h2ch, pidx_r, zb)
    h3, g = _TC_L2(agg2, cnt16, h2, W_l2, bl2, W_r2, batch_p)
    return h3[:N], g


# speedup vs baseline: 2.5666x; 1.7098x over previous
import jax
import jax.numpy as jnp
from jax import lax
from jax.experimental import pallas as pl
from jax.experimental.pallas import tpu as pltpu
from jax.experimental.pallas import tpu_sc as plsc

N = 10000
E = 160000
D_IN = 256
D_H = 512
NG = 16

NT = 16
NPAD = 10240
RPT = NPAD // NT
K = 128
STEPS = 80
EPT = K * STEPS
EPAD = NT * EPT
RB = 256
GRID = NPAD // RB


def _sc_agg_body(y, pidxr, zinit, out, pidx, sb0, sb1, db, rows0, rows1,
                 aggsh, sg0, sg1):
    c = lax.axis_index("c")
    s = lax.axis_index("s")
    r0 = s * RPT
    T = STEPS // 2
    pltpu.sync_copy(pidxr.at[s], pidx)

    def unpack_src(j, buf):
        for i in range(K // 16):
            v = pidx[j, pl.ds(16 * i, 16)]
            buf[pl.ds(16 * i, 16)] = jnp.bitwise_and(v, 0xFFFF)

    def unpack_dst(j, buf):
        for i in range(K // 16):
            v = pidx[j, pl.ds(16 * i, 16)]
            buf[pl.ds(16 * i, 16)] = lax.shift_right_logical(v, 16)

    q = c
    pltpu.sync_copy(zinit.at[pl.ds(r0, RPT)], aggsh.at[pl.ds(r0, RPT)])
    plsc.subcore_barrier()

    unpack_src(0, sb0)
    pltpu.async_copy(y.at[q].at[sb0], rows0, sg0)
    unpack_src(1, sb1)
    pltpu.async_copy(y.at[q].at[sb1], rows1, sg1)

    def step2(t, carry):
        j0 = 2 * t
        more = t + 1 < T
        unpack_dst(j0, db)
        pltpu.make_async_copy(y.at[q].at[sb0], rows0, sg0).wait()
        pltpu.sync_copy(rows0, aggsh.at[db], add=True)

        @pl.when(more)
        def _():
            unpack_src(j0 + 2, sb0)
            pltpu.async_copy(y.at[q].at[sb0], rows0, sg0)
        unpack_dst(j0 + 1, db)
        pltpu.make_async_copy(y.at[q].at[sb1], rows1, sg1).wait()
        pltpu.sync_copy(rows1, aggsh.at[db], add=True)

        @pl.when(more)
        def _():
            unpack_src(j0 + 3, sb1)
            pltpu.async_copy(y.at[q].at[sb1], rows1, sg1)
        return carry
    lax.fori_loop(0, T, step2, 0)
    plsc.subcore_barrier()
    pltpu.sync_copy(aggsh.at[pl.ds(r0, RPT)], out.at[q, pl.ds(r0, RPT)])


S2 = STEPS // 2


def _make_sc_agg0():
    mesh = plsc.VectorSubcoreMesh(core_axis_name="c", subcore_axis_name="s",
                                  num_cores=2, num_subcores=NT)
    out_type = [jax.ShapeDtypeStruct((2, NPAD, 2, 128), jnp.bfloat16)]
    scratch = [
        pltpu.VMEM((S2, K), jnp.int32),
        pltpu.VMEM((K,), jnp.int32),
        pltpu.VMEM((K,), jnp.int32),
        pltpu.VMEM((K,), jnp.int32),
        pltpu.VMEM((K, 2, 128), jnp.bfloat16),
        pltpu.VMEM((K, 2, 128), jnp.bfloat16),
        pltpu.VMEM_SHARED((NPAD, 2, 128), jnp.bfloat16),
        pltpu.SemaphoreType.DMA,
        pltpu.SemaphoreType.DMA,
    ]

    def body(y, pidxr, zinit, out, pidx, sb0, sb1, db, rows0, rows1,
             aggsh, sg0, sg1):
        c = lax.axis_index("c")
        s = lax.axis_index("s")
        r0 = s * RPT
        T = S2 // 2
        pltpu.sync_copy(pidxr.at[c * NT + s], pidx)

        def unpack_src(j, buf):
            for i in range(K // 16):
                v = pidx[j, pl.ds(16 * i, 16)]
                buf[pl.ds(16 * i, 16)] = jnp.bitwise_and(v, 0xFFFF)

        def unpack_dst(j, buf):
            for i in range(K // 16):
                v = pidx[j, pl.ds(16 * i, 16)]
                buf[pl.ds(16 * i, 16)] = lax.shift_right_logical(v, 16)

        pltpu.sync_copy(zinit.at[pl.ds(r0, RPT)], aggsh.at[pl.ds(r0, RPT)])
        plsc.subcore_barrier()

        unpack_src(0, sb0)
        pltpu.async_copy(y.at[sb0], rows0, sg0)
        unpack_src(1, sb1)
        pltpu.async_copy(y.at[sb1], rows1, sg1)

        def step2(t, carry):
            j0 = 2 * t
            more = t + 1 < T
            unpack_dst(j0, db)
            pltpu.make_async_copy(y.at[sb0], rows0, sg0).wait()
            pltpu.sync_copy(rows0, aggsh.at[db], add=True)

            @pl.when(more)
            def _():
                unpack_src(j0 + 2, sb0)
                pltpu.async_copy(y.at[sb0], rows0, sg0)
            unpack_dst(j0 + 1, db)
            pltpu.make_async_copy(y.at[sb1], rows1, sg1).wait()
            pltpu.sync_copy(rows1, aggsh.at[db], add=True)

            @pl.when(more)
            def _():
                unpack_src(j0 + 3, sb1)
                pltpu.async_copy(y.at[sb1], rows1, sg1)
            return carry
        lax.fori_loop(0, T, step2, 0)
        plsc.subcore_barrier()
        pltpu.sync_copy(aggsh.at[pl.ds(r0, RPT)], out.at[c, pl.ds(r0, RPT)])

    return pl.kernel(body, out_type=out_type, mesh=mesh,
                     scratch_types=scratch,
                     compiler_params=pltpu.CompilerParams(
                         use_tc_tiling_on_sc=False))


def _make_sc_agg_bf16():
    mesh = plsc.VectorSubcoreMesh(core_axis_name="c", subcore_axis_name="s",
                                  num_cores=2, num_subcores=NT)
    out_type = [jax.ShapeDtypeStruct((2, NPAD, 2, 128), jnp.bfloat16)]
    scratch = [
        pltpu.VMEM((STEPS, K), jnp.int32),
        pltpu.VMEM((K,), jnp.int32),
        pltpu.VMEM((K,), jnp.int32),
        pltpu.VMEM((K,), jnp.int32),
        pltpu.VMEM((K, 2, 128), jnp.bfloat16),
        pltpu.VMEM((K, 2, 128), jnp.bfloat16),
        pltpu.VMEM_SHARED((NPAD, 2, 128), jnp.bfloat16),
        pltpu.SemaphoreType.DMA,
        pltpu.SemaphoreType.DMA,
    ]

    def body(y, pidxr, zinit, out, pidx, sb0, sb1, db, rows0, rows1,
             aggsh, sg0, sg1):
        _sc_agg_body(y, pidxr, zinit, out, pidx, sb0, sb1, db, rows0, rows1,
                     aggsh, sg0, sg1)

    return pl.kernel(body, out_type=out_type, mesh=mesh,
                     scratch_types=scratch,
                     compiler_params=pltpu.CompilerParams(
                         use_tc_tiling_on_sc=False))


def _make_sc_cnt():
    mesh = plsc.VectorSubcoreMesh(core_axis_name="c", subcore_axis_name="s",
                                  num_cores=2, num_subcores=NT)
    out_type = [jax.ShapeDtypeStruct((2, NPAD, 128), jnp.float32)]
    scratch = [
        pltpu.VMEM((S2, K), jnp.int32),
        pltpu.VMEM((K, 128), jnp.float32),
        pltpu.VMEM_SHARED((NPAD, 128), jnp.float32),
        pltpu.SemaphoreType.DMA,
    ]

    def body(dstr, z128, ones_h, cnt_out, idx_d, ones_v, cntsh, sem):
        c = lax.axis_index("c")
        s = lax.axis_index("s")
        r0 = s * RPT
        pltpu.sync_copy(dstr.at[c * NT + s], idx_d)
        pltpu.sync_copy(ones_h, ones_v)
        pltpu.sync_copy(z128.at[pl.ds(r0, RPT)], cntsh.at[pl.ds(r0, RPT)])
        plsc.subcore_barrier()

        def step(j, carry):
            pltpu.async_copy(ones_v, cntsh.at[idx_d.at[j]], sem, add=True)
            return carry
        lax.fori_loop(0, S2, step, 0)

        def drain(j, carry):
            pltpu.make_async_copy(ones_v, cntsh.at[idx_d.at[j]], sem).wait()
            return carry
        lax.fori_loop(0, S2, drain, 0)
        plsc.subcore_barrier()
        pltpu.sync_copy(cntsh.at[pl.ds(r0, RPT)], cnt_out.at[c, pl.ds(r0, RPT)])

    return pl.kernel(body, out_type=out_type, mesh=mesh,
                     scratch_types=scratch)


_SC_AGG0 = _make_sc_agg0()
_SC_AGG_BF16 = _make_sc_agg_bf16()
_SC_CNT = _make_sc_cnt()


def _make_tc_layer(din, last, agg_bf16):
    nin = din // 128

    def body(*refs):
        if last:
            (agg_ref, cnt_ref, h_ref, wl_ref, b_ref, wr_ref, bt_ref,
             h_out, g_out) = refs
        else:
            (agg_ref, cnt_ref, h_ref, wl_ref, b_ref, wr_ref, a_ref,
             h_out, hch_out) = refs
        if agg_bf16:
            aggf = jnp.concatenate(
                [agg_ref[qq].reshape(RB, din // 2) for qq in range(2)],
                axis=1).astype(jnp.float32)
        else:
            aggf = (agg_ref[0].astype(jnp.float32)
                    + agg_ref[1].astype(jnp.float32)).reshape(RB, din)
        cnt = cnt_ref[0, :, 0:1] + cnt_ref[1, :, 0:1]
        mean = aggf * (1.0 / jnp.maximum(cnt, 1.0))
        out = (jnp.dot(mean, wl_ref[...], preferred_element_type=jnp.float32)
               + b_ref[...]
               + jnp.dot(h_ref[...], wr_ref[...],
                         preferred_element_type=jnp.float32))
        if not last:
            a = a_ref[...]
            out = jnp.where(out >= 0.0, out, a * out)
            h_out[...] = out
            for qq in range(2):
                hch_out[qq] = out[:, qq * 256:(qq + 1) * 256].reshape(
                    RB, 2, 128).astype(jnp.bfloat16)
        else:
            h_out[...] = out
            i = pl.program_id(0)

            @pl.when(i == 0)
            def _():
                g_out[...] = jnp.full((NG, D_H), -jnp.inf, jnp.float32)
            bt = bt_ref[...]
            loc = jnp.stack(
                [jnp.max(jnp.where(bt == gg, out, -jnp.inf), axis=0)
                 for gg in range(NG)], axis=0)
            g_out[...] = jnp.maximum(g_out[...], loc)

    agg_spec = pl.BlockSpec((2, RB, 2, 128), lambda i: (0, i, 0, 0))
    in_specs = [
        agg_spec,
        pl.BlockSpec((2, RB, 128), lambda i: (0, i, 0)),
        pl.BlockSpec((RB, din), lambda i: (i, 0)),
        pl.BlockSpec((din, D_H), lambda i: (0, 0)),
        pl.BlockSpec((1, D_H), lambda i: (0, 0)),
        pl.BlockSpec((din, D_H), lambda i: (0, 0)),
    ]
    if last:
        in_specs.append(pl.BlockSpec((RB, 1), lambda i: (i, 0)))
        out_specs = [
            pl.BlockSpec((RB, D_H), lambda i: (i, 0)),
            pl.BlockSpec((NG, D_H), lambda i: (0, 0)),
        ]
        out_shape = [
            jax.ShapeDtypeStruct((NPAD, D_H), jnp.float32),
            jax.ShapeDtypeStruct((NG, D_H), jnp.float32),
        ]
    else:
        in_specs.append(pl.BlockSpec((1, D_H), lambda i: (0, 0)))
        out_specs = [
            pl.BlockSpec((RB, D_H), lambda i: (i, 0)),
            pl.BlockSpec((2, RB, 2, 128), lambda i: (0, i, 0, 0)),
        ]
        out_shape = [
            jax.ShapeDtypeStruct((NPAD, D_H), jnp.float32),
            jax.ShapeDtypeStruct((2, NPAD, 2, 128), jnp.bfloat16),
        ]

    return pl.pallas_call(
        body,
        grid=(GRID,),
        in_specs=in_specs,
        out_specs=out_specs,
        out_shape=out_shape,
    )


_TC_L0 = _make_tc_layer(D_IN, last=False, agg_bf16=False)
_TC_L1 = _make_tc_layer(D_H, last=False, agg_bf16=True)
_TC_L2 = _make_tc_layer(D_H, last=True, agg_bf16=True)


@jax.jit
def kernel(x, edge_index, batch, W_l0, b_l0, W_r0, W_l1, b_l1, W_r1,
           W_l2, b_l2, W_r2, prelu_a):
    src, dst = edge_index[0], edge_index[1]
    pad_e = EPAD - E
    pad_i = jnp.arange(pad_e, dtype=jnp.int32)
    src_p = jnp.concatenate([src, pad_i % N])
    dst_p = jnp.concatenate([dst, N + (pad_i % (NPAD - N))])
    pidx_p = src_p | (dst_p << 16)
    pidx_r = pidx_p.reshape(EPT, NT).T.reshape(NT, STEPS, K)
    pidx_sp = pidx_p.reshape(EPAD // (2 * NT), 2 * NT).T.reshape(
        2 * NT, S2, K)
    dst_r = dst_p.reshape(EPAD // (2 * NT), 2 * NT).T.reshape(2 * NT, S2, K)

    x_pad = jnp.pad(x, ((0, NPAD - N), (0, 0)))
    x_ch = x_pad.reshape(NPAD, 2, 128).astype(jnp.bfloat16)
    z128 = jnp.zeros((NPAD, 128), jnp.float32)
    zb = jnp.zeros((NPAD, 2, 128), jnp.bfloat16)
    ones128 = jnp.ones((K, 128), jnp.float32)
    batch_p = jnp.pad(batch, (0, NPAD - N),
                      constant_values=NG).reshape(NPAD, 1)

    bl0 = b_l0.reshape(1, D_H)
    bl1 = b_l1.reshape(1, D_H)
    bl2 = b_l2.reshape(1, D_H)
    a_r = prelu_a.reshape(1, D_H)

    (cnt16,) = _SC_CNT(dst_r, z128, ones128)
    (agg0,) = _SC_AGG0(x_ch, pidx_sp, zb)
    h1, h1ch = _TC_L0(agg0, cnt16, x_pad, W_l0, bl0, W_r0, a_r)
    (agg1,) = _SC_AGG_BF16(h1ch, pidx_r, zb)
    h2, h2ch = _TC_L1(agg1, cnt16, h1, W_l1, bl1, W_r1, a_r)
    (agg2,) = _SC_AGG_BF16(h2ch, pidx_r, zb)
    h3, g = _TC_L2(agg2, cnt16, h2, W_l2, bl2, W_r2, batch_p)
    return h3[:N], g
